# Initial kernel scaffold; baseline (speedup 1.0000x reference)
import jax
import jax.numpy as jnp
from jax.experimental import pallas as pl


def kernel(x, edge_index, batch, Wl0, bl0, Wr0, Wl1, bl1, Wr1, Wl2, bl2, Wr2, Wl3, bl3, Wr3, fc1_w, fc1_b, fc2_w, fc2_b):
    raise NotImplementedError("write your pallas kernel here")


# speedup vs baseline: 9.5288x; 9.5288x over previous
import functools

import jax
import jax.numpy as jnp
from jax import lax
from jax.experimental import pallas as pl
from jax.experimental.pallas import tpu as pltpu
from jax.experimental.pallas import tpu_sc as plsc

N_NODES = 10000
PAD_ROWS = 112
N_PAD = N_NODES + PAD_ROWS
HID = 64
G_POOL = 64
NUM_CORES = 2
NUM_SUBCORES = 16
NW = NUM_CORES * NUM_SUBCORES
CHUNK = 128
ROWS_PER_SUB = N_PAD // NUM_SUBCORES


def _sc_agg(nch, with_cnt):
    mesh = plsc.VectorSubcoreMesh(
        core_axis_name="c", subcore_axis_name="s",
        num_cores=NUM_CORES, num_subcores=NUM_SUBCORES)
    agg_type = jax.ShapeDtypeStruct((NUM_CORES, N_PAD, HID), jnp.float32)
    if with_cnt:
        out_type = [agg_type,
                    jax.ShapeDtypeStruct((NUM_CORES * N_PAD,), jnp.float32)]
    else:
        out_type = agg_type
    scratch = [
        pltpu.VMEM((nch, CHUNK), jnp.int32),
        pltpu.VMEM((nch, CHUNK), jnp.int32),
        pltpu.VMEM((CHUNK, HID), jnp.float32),
        pltpu.VMEM_SHARED((N_PAD, HID), jnp.float32),
        pltpu.VMEM_SHARED((N_PAD,), jnp.float32),
        pltpu.VMEM((CHUNK,), jnp.float32),
        pltpu.VMEM((ROWS_PER_SUB,), jnp.float32),
        pltpu.SemaphoreType.DMA,
    ]

    def body(*refs):
        if with_cnt:
            (src_hbm, dst_hbm, p_hbm, z2_hbm, ones_hbm, z1_hbm,
             out_hbm, cnt_hbm,
             src_v, dst_v, rows_v, agg_sh, cnt_sh, ones_v, cntb_v,
             sem) = refs
        else:
            (src_hbm, dst_hbm, p_hbm, z2_hbm,
             out_hbm,
             src_v, dst_v, rows_v, agg_sh, cnt_sh, ones_v, cntb_v,
             sem) = refs
        cid = lax.axis_index("c")
        sid = lax.axis_index("s")
        wid = cid * NUM_SUBCORES + sid
        rbase = sid * ROWS_PER_SUB
        pltpu.sync_copy(z2_hbm.at[pl.ds(rbase, ROWS_PER_SUB)],
                        agg_sh.at[pl.ds(rbase, ROWS_PER_SUB)])
        if with_cnt:
            pltpu.sync_copy(z1_hbm.at[pl.ds(rbase, ROWS_PER_SUB)], cntb_v)
            pltpu.sync_copy(cntb_v, cnt_sh.at[pl.ds(rbase, ROWS_PER_SUB)])
            pltpu.sync_copy(ones_hbm, ones_v)
        pltpu.sync_copy(src_hbm.at[wid], src_v)
        pltpu.sync_copy(dst_hbm.at[wid], dst_v)
        plsc.subcore_barrier()

        def step(j, carry):
            pltpu.async_copy(p_hbm.at[src_v.at[j]], rows_v, sem).wait()
            pltpu.sync_copy(rows_v, agg_sh.at[dst_v.at[j]], add=True)
            if with_cnt:
                pltpu.sync_copy(ones_v, cnt_sh.at[dst_v.at[j]], add=True)
            return carry

        lax.fori_loop(0, nch, step, 0)
        plsc.subcore_barrier()
        pltpu.sync_copy(agg_sh.at[pl.ds(rbase, ROWS_PER_SUB)],
                        out_hbm.at[cid, pl.ds(rbase, ROWS_PER_SUB)])
        if with_cnt:
            pltpu.sync_copy(cnt_sh.at[pl.ds(rbase, ROWS_PER_SUB)], cntb_v)
            pltpu.sync_copy(cntb_v,
                            cnt_hbm.at[pl.ds(cid * N_PAD + rbase,
                                             ROWS_PER_SUB)])

    return pl.kernel(body, out_type=out_type, mesh=mesh,
                     scratch_types=scratch,
                     compiler_params=pltpu.CompilerParams(
                         use_tc_tiling_on_sc=False))


def _tc_pre(x_ref, wl_ref, wr_ref, bl_ref, p_ref, r_ref):
    x = x_ref[...]
    p_ref[:N_NODES, :] = jnp.dot(x, wl_ref[...],
                                 preferred_element_type=jnp.float32)
    p_ref[N_NODES:, :] = jnp.zeros((PAD_ROWS, HID), jnp.float32)
    r_ref[...] = jnp.dot(x, wr_ref[...],
                         preferred_element_type=jnp.float32) + bl_ref[...]


def _tc_mid_first(a_ref, cnt0_ref, cnt1_ref, r_ref, wl_ref, wr_ref, bl_ref,
                  h_ref, p_ref, rn_ref, inv_ref):
    cnt = cnt0_ref[:N_NODES, :] + cnt1_ref[:N_NODES, :]
    inv = 1.0 / jnp.maximum(cnt, 1.0)
    inv_ref[...] = inv
    agg = a_ref[0, :N_NODES, :] + a_ref[1, :N_NODES, :]
    h = agg * inv + r_ref[...]
    h_ref[...] = h
    p_ref[:N_NODES, :] = jnp.dot(h, wl_ref[...],
                                 preferred_element_type=jnp.float32)
    p_ref[N_NODES:, :] = jnp.zeros((PAD_ROWS, HID), jnp.float32)
    rn_ref[...] = jnp.dot(h, wr_ref[...],
                          preferred_element_type=jnp.float32) + bl_ref[...]


def _tc_mid(a_ref, inv_ref, r_ref, wl_ref, wr_ref, bl_ref,
            h_ref, p_ref, rn_ref):
    agg = a_ref[0, :N_NODES, :] + a_ref[1, :N_NODES, :]
    h = agg * inv_ref[...] + r_ref[...]
    h_ref[...] = h
    p_ref[:N_NODES, :] = jnp.dot(h, wl_ref[...],
                                 preferred_element_type=jnp.float32)
    p_ref[N_NODES:, :] = jnp.zeros((PAD_ROWS, HID), jnp.float32)
    rn_ref[...] = jnp.dot(h, wr_ref[...],
                          preferred_element_type=jnp.float32) + bl_ref[...]


def _tc_final(a_ref, inv_ref, r_ref, h1_ref, h2_ref, h3_ref, batch_ref,
              fc1w_ref, fc1b_ref, fc2w_ref, fc2b_ref, out_ref):
    agg = a_ref[0, :N_NODES, :] + a_ref[1, :N_NODES, :]
    h4 = agg * inv_ref[...] + r_ref[...]
    b = batch_ref[...]
    gids = lax.broadcasted_iota(jnp.int32, (G_POOL, N_NODES), 0)
    mask = (b == gids).astype(jnp.float32)
    s1 = jnp.dot(mask, h1_ref[...], preferred_element_type=jnp.float32)
    s2 = jnp.dot(mask, h2_ref[...], preferred_element_type=jnp.float32)
    s3 = jnp.dot(mask, h3_ref[...], preferred_element_type=jnp.float32)
    s4 = jnp.dot(mask, h4, preferred_element_type=jnp.float32)
    cnt = jnp.sum(mask, axis=1, keepdims=True)
    pooled = jnp.concatenate([s1, s2, s3, s4], axis=1) / jnp.maximum(cnt, 1.0)
    hh = jnp.maximum(
        jnp.dot(pooled, fc1w_ref[...], preferred_element_type=jnp.float32)
        + fc1b_ref[...], 0.0)
    out_ref[...] = jnp.dot(hh, fc2w_ref[...],
                           preferred_element_type=jnp.float32) + fc2b_ref[...]


def _tc_call(body, out_shapes):
    return pl.pallas_call(body, out_shape=out_shapes)


def kernel(x, edge_index, batch, Wl0, bl0, Wr0, Wl1, bl1, Wr1, Wl2, bl2, Wr2,
           Wl3, bl3, Wr3, fc1_w, fc1_b, fc2_w, fc2_b):
    E = edge_index.shape[1]
    nch = -(-E // (NW * CHUNK))
    epad = nch * NW * CHUNK
    pad = epad - E
    src = edge_index[0]
    dst = edge_index[1]
    if pad:
        pad_idx = (N_NODES
                   + (jnp.arange(pad, dtype=jnp.int32) % PAD_ROWS))
        src = jnp.concatenate([src.astype(jnp.int32), pad_idx])
        dst = jnp.concatenate([dst.astype(jnp.int32), pad_idx])
    src3 = src.reshape(NW, nch, CHUNK)
    dst3 = dst.reshape(NW, nch, CHUNK)

    zeros2d = jnp.zeros((N_PAD, HID), jnp.float32)
    zeros1d = jnp.zeros((N_PAD,), jnp.float32)
    ones_c = jnp.ones((CHUNK,), jnp.float32)

    nhid = jax.ShapeDtypeStruct((N_NODES, HID), jnp.float32)
    phid = jax.ShapeDtypeStruct((N_PAD, HID), jnp.float32)
    ninv = jax.ShapeDtypeStruct((N_NODES, 1), jnp.float32)

    sc_first = _sc_agg(nch, with_cnt=True)
    sc_rest = _sc_agg(nch, with_cnt=False)

    p0, r0 = _tc_call(_tc_pre, [phid, nhid])(
        x, Wl0, Wr0, bl0.reshape(1, HID))
    a0, cntp = sc_first(src3, dst3, p0, zeros2d, ones_c, zeros1d)
    cntp = cntp.reshape(NUM_CORES, N_PAD)
    cnt0 = cntp[0].reshape(N_PAD, 1)
    cnt1 = cntp[1].reshape(N_PAD, 1)
    h1, p1, r1, inv = _tc_call(_tc_mid_first, [nhid, phid, nhid, ninv])(
        a0, cnt0, cnt1, r0, Wl1, Wr1, bl1.reshape(1, HID))
    a1 = sc_rest(src3, dst3, p1, zeros2d)
    h2, p2, r2 = _tc_call(_tc_mid, [nhid, phid, nhid])(
        a1, inv, r1, Wl2, Wr2, bl2.reshape(1, HID))
    a2 = sc_rest(src3, dst3, p2, zeros2d)
    h3, p3, r3 = _tc_call(_tc_mid, [nhid, phid, nhid])(
        a2, inv, r2, Wl3, Wr3, bl3.reshape(1, HID))
    a3 = sc_rest(src3, dst3, p3, zeros2d)
    out = _tc_call(_tc_final,
                   jax.ShapeDtypeStruct((G_POOL, fc2_w.shape[1]),
                                        jnp.float32))(
        a3, inv, r3, h1, h2, h3, batch.astype(jnp.int32).reshape(1, N_NODES),
        fc1_w, fc1_b.reshape(1, HID), fc2_w, fc2_b.reshape(1, fc2_w.shape[1]))
    return out


# speedup vs baseline: 12.2323x; 1.2837x over previous
import functools

import jax
import jax.numpy as jnp
from jax import lax
from jax.experimental import pallas as pl
from jax.experimental.pallas import tpu as pltpu
from jax.experimental.pallas import tpu_sc as plsc

N_NODES = 10000
PAD_ROWS = 112
N_PAD = N_NODES + PAD_ROWS
HID = 64
G_POOL = 64
NUM_CORES = 2
NUM_SUBCORES = 16
NW = NUM_CORES * NUM_SUBCORES
CHUNK = 128
GRP = 8
ROWS_PER_SUB = N_PAD // NUM_SUBCORES


def _sc_agg(nch, with_cnt):
    assert nch % GRP == 0
    mesh = plsc.VectorSubcoreMesh(
        core_axis_name="c", subcore_axis_name="s",
        num_cores=NUM_CORES, num_subcores=NUM_SUBCORES)
    agg_type = jax.ShapeDtypeStruct((NUM_CORES, N_PAD, HID), jnp.float32)
    if with_cnt:
        out_type = [agg_type,
                    jax.ShapeDtypeStruct((NUM_CORES * N_PAD,), jnp.float32)]
    else:
        out_type = agg_type
    scratch = [
        pltpu.VMEM((nch + GRP, CHUNK), jnp.int32),
        pltpu.VMEM((nch + GRP, CHUNK), jnp.int32),
        [pltpu.VMEM((CHUNK, HID), jnp.float32) for _ in range(GRP)],
        pltpu.VMEM_SHARED((N_PAD, HID), jnp.float32),
        pltpu.VMEM_SHARED((N_PAD,), jnp.float32),
        pltpu.VMEM((CHUNK,), jnp.float32),
        pltpu.VMEM((ROWS_PER_SUB,), jnp.float32),
        [pltpu.SemaphoreType.DMA for _ in range(GRP)],
        [pltpu.SemaphoreType.DMA for _ in range(GRP)],
        [pltpu.SemaphoreType.DMA for _ in range(GRP)],
    ]

    def body(*refs):
        if with_cnt:
            (src_hbm, dst_hbm, p_hbm, z2_hbm, ones_hbm, z1_hbm,
             out_hbm, cnt_hbm,
             src_v, dst_v, rows, agg_sh, cnt_sh, ones_v, cntb_v,
             gsem, ssem, csem) = refs
        else:
            (src_hbm, dst_hbm, p_hbm, z2_hbm,
             out_hbm,
             src_v, dst_v, rows, agg_sh, cnt_sh, ones_v, cntb_v,
             gsem, ssem, csem) = refs
        cid = lax.axis_index("c")
        sid = lax.axis_index("s")
        wid = cid * NUM_SUBCORES + sid
        rbase = sid * ROWS_PER_SUB
        pltpu.sync_copy(z2_hbm.at[pl.ds(rbase, ROWS_PER_SUB)],
                        agg_sh.at[pl.ds(rbase, ROWS_PER_SUB)])
        if with_cnt:
            pltpu.sync_copy(z1_hbm.at[pl.ds(rbase, ROWS_PER_SUB)], cntb_v)
            pltpu.sync_copy(cntb_v, cnt_sh.at[pl.ds(rbase, ROWS_PER_SUB)])
            pltpu.sync_copy(ones_hbm, ones_v)
        pltpu.sync_copy(src_hbm.at[wid], src_v)
        pltpu.sync_copy(dst_hbm.at[wid], dst_v)
        plsc.subcore_barrier()

        def gather_start(j, b):
            pltpu.async_copy(p_hbm.at[src_v.at[j]], rows[b], gsem[b])

        def gather_wait(j, b):
            pltpu.make_async_copy(p_hbm.at[src_v.at[j]], rows[b],
                                  gsem[b]).wait()

        ngr = nch // GRP
        for b in range(GRP):
            gather_start(b, b)

        def group(g, carry):
            base = g * GRP
            for b in range(GRP):
                gather_wait(base + b, b)
                pltpu.async_copy(rows[b], agg_sh.at[dst_v.at[base + b]],
                                 ssem[b], add=True)
                if with_cnt:
                    pltpu.async_copy(ones_v, cnt_sh.at[dst_v.at[base + b]],
                                     csem[b], add=True)
            for b in range(GRP):
                pltpu.make_async_copy(rows[b], agg_sh.at[dst_v.at[base + b]],
                                      ssem[b]).wait()
                if with_cnt:
                    pltpu.make_async_copy(ones_v,
                                          cnt_sh.at[dst_v.at[base + b]],
                                          csem[b]).wait()
                gather_start(base + GRP + b, b)
            return carry

        lax.fori_loop(0, ngr, group, 0)
        for b in range(GRP):
            gather_wait(ngr * GRP + b, b)
        plsc.subcore_barrier()
        pltpu.sync_copy(agg_sh.at[pl.ds(rbase, ROWS_PER_SUB)],
                        out_hbm.at[cid, pl.ds(rbase, ROWS_PER_SUB)])
        if with_cnt:
            pltpu.sync_copy(cnt_sh.at[pl.ds(rbase, ROWS_PER_SUB)], cntb_v)
            pltpu.sync_copy(cntb_v,
                            cnt_hbm.at[pl.ds(cid * N_PAD + rbase,
                                             ROWS_PER_SUB)])

    return pl.kernel(body, out_type=out_type, mesh=mesh,
                     scratch_types=scratch,
                     compiler_params=pltpu.CompilerParams(
                         use_tc_tiling_on_sc=False))


def _tc_pre(x_ref, wl_ref, wr_ref, bl_ref, p_ref, r_ref):
    x = x_ref[...]
    p_ref[:N_NODES, :] = jnp.dot(x, wl_ref[...],
                                 preferred_element_type=jnp.float32)
    p_ref[N_NODES:, :] = jnp.zeros((PAD_ROWS, HID), jnp.float32)
    r_ref[...] = jnp.dot(x, wr_ref[...],
                         preferred_element_type=jnp.float32) + bl_ref[...]


def _tc_mid_first(a_ref, cnt0_ref, cnt1_ref, r_ref, wl_ref, wr_ref, bl_ref,
                  h_ref, p_ref, rn_ref, inv_ref):
    cnt = cnt0_ref[:N_NODES, :] + cnt1_ref[:N_NODES, :]
    inv = 1.0 / jnp.maximum(cnt, 1.0)
    inv_ref[...] = inv
    agg = a_ref[0, :N_NODES, :] + a_ref[1, :N_NODES, :]
    h = agg * inv + r_ref[...]
    h_ref[...] = h
    p_ref[:N_NODES, :] = jnp.dot(h, wl_ref[...],
                                 preferred_element_type=jnp.float32)
    p_ref[N_NODES:, :] = jnp.zeros((PAD_ROWS, HID), jnp.float32)
    rn_ref[...] = jnp.dot(h, wr_ref[...],
                          preferred_element_type=jnp.float32) + bl_ref[...]


def _tc_mid(a_ref, inv_ref, r_ref, wl_ref, wr_ref, bl_ref,
            h_ref, p_ref, rn_ref):
    agg = a_ref[0, :N_NODES, :] + a_ref[1, :N_NODES, :]
    h = agg * inv_ref[...] + r_ref[...]
    h_ref[...] = h
    p_ref[:N_NODES, :] = jnp.dot(h, wl_ref[...],
                                 preferred_element_type=jnp.float32)
    p_ref[N_NODES:, :] = jnp.zeros((PAD_ROWS, HID), jnp.float32)
    rn_ref[...] = jnp.dot(h, wr_ref[...],
                          preferred_element_type=jnp.float32) + bl_ref[...]


def _tc_final(a_ref, inv_ref, r_ref, h1_ref, h2_ref, h3_ref, batch_ref,
              fc1w_ref, fc1b_ref, fc2w_ref, fc2b_ref, out_ref):
    agg = a_ref[0, :N_NODES, :] + a_ref[1, :N_NODES, :]
    h4 = agg * inv_ref[...] + r_ref[...]
    b = batch_ref[...]
    gids = lax.broadcasted_iota(jnp.int32, (G_POOL, N_NODES), 0)
    mask = (b == gids).astype(jnp.float32)
    s1 = jnp.dot(mask, h1_ref[...], preferred_element_type=jnp.float32)
    s2 = jnp.dot(mask, h2_ref[...], preferred_element_type=jnp.float32)
    s3 = jnp.dot(mask, h3_ref[...], preferred_element_type=jnp.float32)
    s4 = jnp.dot(mask, h4, preferred_element_type=jnp.float32)
    cnt = jnp.sum(mask, axis=1, keepdims=True)
    pooled = jnp.concatenate([s1, s2, s3, s4], axis=1) / jnp.maximum(cnt, 1.0)
    hh = jnp.maximum(
        jnp.dot(pooled, fc1w_ref[...], preferred_element_type=jnp.float32)
        + fc1b_ref[...], 0.0)
    out_ref[...] = jnp.dot(hh, fc2w_ref[...],
                           preferred_element_type=jnp.float32) + fc2b_ref[...]


def _tc_call(body, out_shapes):
    return pl.pallas_call(body, out_shape=out_shapes)


def kernel(x, edge_index, batch, Wl0, bl0, Wr0, Wl1, bl1, Wr1, Wl2, bl2, Wr2,
           Wl3, bl3, Wr3, fc1_w, fc1_b, fc2_w, fc2_b):
    E = edge_index.shape[1]
    nch = -(-E // (NW * CHUNK))
    nch = -(-nch // GRP) * GRP
    epad = nch * NW * CHUNK
    pad = epad - E
    src = edge_index[0]
    dst = edge_index[1]
    if pad:
        pad_idx = (N_NODES
                   + (jnp.arange(pad, dtype=jnp.int32) % PAD_ROWS))
        src = jnp.concatenate([src.astype(jnp.int32), pad_idx])
        dst = jnp.concatenate([dst.astype(jnp.int32), pad_idx])
    junk = N_NODES + (jnp.arange(CHUNK, dtype=jnp.int32) % PAD_ROWS)
    junk3 = jnp.broadcast_to(junk, (NW, GRP, CHUNK))
    src3 = jnp.concatenate([src.reshape(NW, nch, CHUNK), junk3], axis=1)
    dst3 = jnp.concatenate([dst.reshape(NW, nch, CHUNK), junk3], axis=1)

    zeros2d = jnp.zeros((N_PAD, HID), jnp.float32)
    zeros1d = jnp.zeros((N_PAD,), jnp.float32)
    ones_c = jnp.ones((CHUNK,), jnp.float32)

    nhid = jax.ShapeDtypeStruct((N_NODES, HID), jnp.float32)
    phid = jax.ShapeDtypeStruct((N_PAD, HID), jnp.float32)
    ninv = jax.ShapeDtypeStruct((N_NODES, 1), jnp.float32)

    sc_first = _sc_agg(nch, with_cnt=True)
    sc_rest = _sc_agg(nch, with_cnt=False)

    p0, r0 = _tc_call(_tc_pre, [phid, nhid])(
        x, Wl0, Wr0, bl0.reshape(1, HID))
    a0, cntp = sc_first(src3, dst3, p0, zeros2d, ones_c, zeros1d)
    cntp = cntp.reshape(NUM_CORES, N_PAD)
    cnt0 = cntp[0].reshape(N_PAD, 1)
    cnt1 = cntp[1].reshape(N_PAD, 1)
    h1, p1, r1, inv = _tc_call(_tc_mid_first, [nhid, phid, nhid, ninv])(
        a0, cnt0, cnt1, r0, Wl1, Wr1, bl1.reshape(1, HID))
    a1 = sc_rest(src3, dst3, p1, zeros2d)
    h2, p2, r2 = _tc_call(_tc_mid, [nhid, phid, nhid])(
        a1, inv, r1, Wl2, Wr2, bl2.reshape(1, HID))
    a2 = sc_rest(src3, dst3, p2, zeros2d)
    h3, p3, r3 = _tc_call(_tc_mid, [nhid, phid, nhid])(
        a2, inv, r2, Wl3, Wr3, bl3.reshape(1, HID))
    a3 = sc_rest(src3, dst3, p3, zeros2d)
    out = _tc_call(_tc_final,
                   jax.ShapeDtypeStruct((G_POOL, fc2_w.shape[1]),
                                        jnp.float32))(
        a3, inv, r3, h1, h2, h3, batch.astype(jnp.int32).reshape(1, N_NODES),
        fc1_w, fc1_b.reshape(1, HID), fc2_w, fc2_b.reshape(1, fc2_w.shape[1]))
    return out


# speedup vs baseline: 12.2757x; 1.0035x over previous
import functools

import jax
import jax.numpy as jnp
from jax import lax
from jax.experimental import pallas as pl
from jax.experimental.pallas import tpu as pltpu
from jax.experimental.pallas import tpu_sc as plsc

N_NODES = 10000
PAD_ROWS = 112
N_PAD = N_NODES + PAD_ROWS
HID = 64
G_POOL = 64
NUM_CORES = 2
NUM_SUBCORES = 16
NW = NUM_CORES * NUM_SUBCORES
CHUNK = 128
GRP = 8
ROWS_PER_SUB = N_PAD // NUM_SUBCORES


def _sc_agg(nch, with_cnt):
    assert nch % GRP == 0
    mesh = plsc.VectorSubcoreMesh(
        core_axis_name="c", subcore_axis_name="s",
        num_cores=NUM_CORES, num_subcores=NUM_SUBCORES)
    agg_type = jax.ShapeDtypeStruct((NUM_CORES, N_PAD, HID), jnp.float32)
    if with_cnt:
        out_type = [agg_type,
                    jax.ShapeDtypeStruct((NUM_CORES * N_PAD,), jnp.float32)]
    else:
        out_type = agg_type
    scratch = [
        pltpu.VMEM((nch + GRP, CHUNK), jnp.int32),
        pltpu.VMEM((nch + GRP, CHUNK), jnp.int32),
        [pltpu.VMEM((CHUNK, HID), jnp.float32) for _ in range(GRP)],
        pltpu.VMEM_SHARED((N_PAD, HID), jnp.float32),
        pltpu.VMEM_SHARED((N_PAD,), jnp.float32),
        pltpu.VMEM((CHUNK,), jnp.float32),
        pltpu.VMEM((ROWS_PER_SUB,), jnp.float32),
        [pltpu.SemaphoreType.DMA for _ in range(GRP)],
        [pltpu.SemaphoreType.DMA for _ in range(GRP)],
        [pltpu.SemaphoreType.DMA for _ in range(GRP)],
    ]

    def body(*refs):
        if with_cnt:
            (src_hbm, dst_hbm, p_hbm, z2_hbm, ones_hbm, z1_hbm,
             out_hbm, cnt_hbm,
             src_v, dst_v, rows, agg_sh, cnt_sh, ones_v, cntb_v,
             gsem, ssem, csem) = refs
        else:
            (src_hbm, dst_hbm, p_hbm, z2_hbm,
             out_hbm,
             src_v, dst_v, rows, agg_sh, cnt_sh, ones_v, cntb_v,
             gsem, ssem, csem) = refs
        cid = lax.axis_index("c")
        sid = lax.axis_index("s")
        wid = cid * NUM_SUBCORES + sid
        rbase = sid * ROWS_PER_SUB
        def gather_start(j, b):
            pltpu.async_copy(p_hbm.at[src_v.at[j]], rows[b], gsem[b])

        def gather_wait(j, b):
            pltpu.make_async_copy(p_hbm.at[src_v.at[j]], rows[b],
                                  gsem[b]).wait()

        ngr = nch // GRP
        pltpu.sync_copy(src_hbm.at[wid], src_v)
        for b in range(GRP):
            gather_start(b, b)
        pltpu.sync_copy(dst_hbm.at[wid], dst_v)
        pltpu.sync_copy(z2_hbm.at[pl.ds(rbase, ROWS_PER_SUB)],
                        agg_sh.at[pl.ds(rbase, ROWS_PER_SUB)])
        if with_cnt:
            pltpu.sync_copy(z1_hbm.at[pl.ds(rbase, ROWS_PER_SUB)], cntb_v)
            pltpu.sync_copy(cntb_v, cnt_sh.at[pl.ds(rbase, ROWS_PER_SUB)])
            pltpu.sync_copy(ones_hbm, ones_v)
        plsc.subcore_barrier()

        def group(g, carry):
            base = g * GRP
            for b in range(GRP):
                gather_wait(base + b, b)
                pltpu.async_copy(rows[b], agg_sh.at[dst_v.at[base + b]],
                                 ssem[b], add=True)
                if with_cnt:
                    pltpu.async_copy(ones_v, cnt_sh.at[dst_v.at[base + b]],
                                     csem[b], add=True)
            for b in range(GRP):
                pltpu.make_async_copy(rows[b], agg_sh.at[dst_v.at[base + b]],
                                      ssem[b]).wait()
                if with_cnt:
                    pltpu.make_async_copy(ones_v,
                                          cnt_sh.at[dst_v.at[base + b]],
                                          csem[b]).wait()
                gather_start(base + GRP + b, b)
            return carry

        lax.fori_loop(0, ngr, group, 0)
        for b in range(GRP):
            gather_wait(ngr * GRP + b, b)
        plsc.subcore_barrier()
        pltpu.sync_copy(agg_sh.at[pl.ds(rbase, ROWS_PER_SUB)],
                        out_hbm.at[cid, pl.ds(rbase, ROWS_PER_SUB)])
        if with_cnt:
            pltpu.sync_copy(cnt_sh.at[pl.ds(rbase, ROWS_PER_SUB)], cntb_v)
            pltpu.sync_copy(cntb_v,
                            cnt_hbm.at[pl.ds(cid * N_PAD + rbase,
                                             ROWS_PER_SUB)])

    return pl.kernel(body, out_type=out_type, mesh=mesh,
                     scratch_types=scratch,
                     compiler_params=pltpu.CompilerParams(
                         use_tc_tiling_on_sc=False))


def _tc_pre(x_ref, wl_ref, wr_ref, bl_ref, p_ref, r_ref):
    x = x_ref[...]
    p_ref[:N_NODES, :] = jnp.dot(x, wl_ref[...],
                                 preferred_element_type=jnp.float32)
    p_ref[N_NODES:, :] = jnp.zeros((PAD_ROWS, HID), jnp.float32)
    r_ref[...] = jnp.dot(x, wr_ref[...],
                         preferred_element_type=jnp.float32) + bl_ref[...]


def _tc_mid_first(a_ref, cnt0_ref, cnt1_ref, r_ref, wl_ref, wr_ref, bl_ref,
                  h_ref, p_ref, rn_ref, inv_ref):
    cnt = cnt0_ref[:N_NODES, :] + cnt1_ref[:N_NODES, :]
    inv = 1.0 / jnp.maximum(cnt, 1.0)
    inv_ref[...] = inv
    agg = a_ref[0, :N_NODES, :] + a_ref[1, :N_NODES, :]
    h = agg * inv + r_ref[...]
    h_ref[...] = h
    p_ref[:N_NODES, :] = jnp.dot(h, wl_ref[...],
                                 preferred_element_type=jnp.float32)
    p_ref[N_NODES:, :] = jnp.zeros((PAD_ROWS, HID), jnp.float32)
    rn_ref[...] = jnp.dot(h, wr_ref[...],
                          preferred_element_type=jnp.float32) + bl_ref[...]


def _tc_mid(a_ref, inv_ref, r_ref, wl_ref, wr_ref, bl_ref,
            h_ref, p_ref, rn_ref):
    agg = a_ref[0, :N_NODES, :] + a_ref[1, :N_NODES, :]
    h = agg * inv_ref[...] + r_ref[...]
    h_ref[...] = h
    p_ref[:N_NODES, :] = jnp.dot(h, wl_ref[...],
                                 preferred_element_type=jnp.float32)
    p_ref[N_NODES:, :] = jnp.zeros((PAD_ROWS, HID), jnp.float32)
    rn_ref[...] = jnp.dot(h, wr_ref[...],
                          preferred_element_type=jnp.float32) + bl_ref[...]


def _tc_final(a_ref, inv_ref, r_ref, h1_ref, h2_ref, h3_ref, batch_ref,
              fc1w_ref, fc1b_ref, fc2w_ref, fc2b_ref, out_ref):
    agg = a_ref[0, :N_NODES, :] + a_ref[1, :N_NODES, :]
    h4 = agg * inv_ref[...] + r_ref[...]
    b = batch_ref[...]
    gids = lax.broadcasted_iota(jnp.int32, (G_POOL, N_NODES), 0)
    mask = (b == gids).astype(jnp.float32)
    s1 = jnp.dot(mask, h1_ref[...], preferred_element_type=jnp.float32)
    s2 = jnp.dot(mask, h2_ref[...], preferred_element_type=jnp.float32)
    s3 = jnp.dot(mask, h3_ref[...], preferred_element_type=jnp.float32)
    s4 = jnp.dot(mask, h4, preferred_element_type=jnp.float32)
    cnt = jnp.sum(mask, axis=1, keepdims=True)
    pooled = jnp.concatenate([s1, s2, s3, s4], axis=1) / jnp.maximum(cnt, 1.0)
    hh = jnp.maximum(
        jnp.dot(pooled, fc1w_ref[...], preferred_element_type=jnp.float32)
        + fc1b_ref[...], 0.0)
    out_ref[...] = jnp.dot(hh, fc2w_ref[...],
                           preferred_element_type=jnp.float32) + fc2b_ref[...]


def _tc_call(body, out_shapes):
    return pl.pallas_call(body, out_shape=out_shapes)


def kernel(x, edge_index, batch, Wl0, bl0, Wr0, Wl1, bl1, Wr1, Wl2, bl2, Wr2,
           Wl3, bl3, Wr3, fc1_w, fc1_b, fc2_w, fc2_b):
    E = edge_index.shape[1]
    nch = -(-E // (NW * CHUNK))
    nch = -(-nch // GRP) * GRP
    epad = nch * NW * CHUNK
    pad = epad - E
    src = edge_index[0]
    dst = edge_index[1]
    if pad:
        pad_idx = (N_NODES
                   + (jnp.arange(pad, dtype=jnp.int32) % PAD_ROWS))
        src = jnp.concatenate([src.astype(jnp.int32), pad_idx])
        dst = jnp.concatenate([dst.astype(jnp.int32), pad_idx])
    junk = N_NODES + (jnp.arange(CHUNK, dtype=jnp.int32) % PAD_ROWS)
    junk3 = jnp.broadcast_to(junk, (NW, GRP, CHUNK))
    src3 = jnp.concatenate([src.reshape(NW, nch, CHUNK), junk3], axis=1)
    dst3 = jnp.concatenate([dst.reshape(NW, nch, CHUNK), junk3], axis=1)

    zeros2d = jnp.zeros((N_PAD, HID), jnp.float32)
    zeros1d = jnp.zeros((N_PAD,), jnp.float32)
    ones_c = jnp.ones((CHUNK,), jnp.float32)

    nhid = jax.ShapeDtypeStruct((N_NODES, HID), jnp.float32)
    phid = jax.ShapeDtypeStruct((N_PAD, HID), jnp.float32)
    ninv = jax.ShapeDtypeStruct((N_NODES, 1), jnp.float32)

    sc_first = _sc_agg(nch, with_cnt=True)
    sc_rest = _sc_agg(nch, with_cnt=False)

    p0, r0 = _tc_call(_tc_pre, [phid, nhid])(
        x, Wl0, Wr0, bl0.reshape(1, HID))
    a0, cntp = sc_first(src3, dst3, p0, zeros2d, ones_c, zeros1d)
    cntp = cntp.reshape(NUM_CORES, N_PAD)
    cnt0 = cntp[0].reshape(N_PAD, 1)
    cnt1 = cntp[1].reshape(N_PAD, 1)
    h1, p1, r1, inv = _tc_call(_tc_mid_first, [nhid, phid, nhid, ninv])(
        a0, cnt0, cnt1, r0, Wl1, Wr1, bl1.reshape(1, HID))
    a1 = sc_rest(src3, dst3, p1, zeros2d)
    h2, p2, r2 = _tc_call(_tc_mid, [nhid, phid, nhid])(
        a1, inv, r1, Wl2, Wr2, bl2.reshape(1, HID))
    a2 = sc_rest(src3, dst3, p2, zeros2d)
    h3, p3, r3 = _tc_call(_tc_mid, [nhid, phid, nhid])(
        a2, inv, r2, Wl3, Wr3, bl3.reshape(1, HID))
    a3 = sc_rest(src3, dst3, p3, zeros2d)
    out = _tc_call(_tc_final,
                   jax.ShapeDtypeStruct((G_POOL, fc2_w.shape[1]),
                                        jnp.float32))(
        a3, inv, r3, h1, h2, h3, batch.astype(jnp.int32).reshape(1, N_NODES),
        fc1_w, fc1_b.reshape(1, HID), fc2_w, fc2_b.reshape(1, fc2_w.shape[1]))
    return out


# speedup vs baseline: 12.2955x; 1.0016x over previous
import jax
import jax.numpy as jnp
from jax import lax
from jax.experimental import pallas as pl
from jax.experimental.pallas import tpu as pltpu
from jax.experimental.pallas import tpu_sc as plsc

N_NODES = 10000
PAD_ROWS = 112
N_PAD = N_NODES + PAD_ROWS
HID = 64
G_POOL = 64
NUM_CORES = 2
NUM_SUBCORES = 16
NW = NUM_CORES * NUM_SUBCORES
CHUNK = 128
GRP = 8
ROWS_PER_SUB = N_PAD // NUM_SUBCORES


def _sc_agg(nch, with_cnt):
    assert nch % GRP == 0
    mesh = plsc.VectorSubcoreMesh(
        core_axis_name="c", subcore_axis_name="s",
        num_cores=NUM_CORES, num_subcores=NUM_SUBCORES)
    agg_type = jax.ShapeDtypeStruct((NUM_CORES, N_PAD, HID), jnp.float32)
    if with_cnt:
        out_type = [agg_type,
                    jax.ShapeDtypeStruct((NUM_CORES * N_PAD,), jnp.float32)]
    else:
        out_type = agg_type
    scratch = [
        pltpu.VMEM((nch + GRP, CHUNK), jnp.int32),
        pltpu.VMEM((nch + GRP, CHUNK), jnp.int32),
        [pltpu.VMEM((CHUNK, HID), jnp.float32) for _ in range(GRP)],
        pltpu.VMEM_SHARED((N_PAD, HID), jnp.float32),
        pltpu.VMEM_SHARED((N_PAD,), jnp.float32),
        pltpu.VMEM((CHUNK,), jnp.float32),
        pltpu.VMEM((ROWS_PER_SUB,), jnp.float32),
        [pltpu.SemaphoreType.DMA for _ in range(GRP)],
        [pltpu.SemaphoreType.DMA for _ in range(GRP)],
        [pltpu.SemaphoreType.DMA for _ in range(GRP)],
    ]

    def body(*refs):
        if with_cnt:
            (src_hbm, dst_hbm, p_hbm, z2_hbm, ones_hbm, z1_hbm,
             out_hbm, cnt_hbm,
             src_v, dst_v, rows, agg_sh, cnt_sh, ones_v, cntb_v,
             gsem, ssem, csem) = refs
        else:
            (src_hbm, dst_hbm, p_hbm, z2_hbm,
             out_hbm,
             src_v, dst_v, rows, agg_sh, cnt_sh, ones_v, cntb_v,
             gsem, ssem, csem) = refs
        cid = lax.axis_index("c")
        sid = lax.axis_index("s")
        wid = cid * NUM_SUBCORES + sid
        rbase = sid * ROWS_PER_SUB
        def gather_start(j, b):
            pltpu.async_copy(p_hbm.at[src_v.at[j]], rows[b], gsem[b])

        def gather_wait(j, b):
            pltpu.make_async_copy(p_hbm.at[src_v.at[j]], rows[b],
                                  gsem[b]).wait()

        ngr = nch // GRP
        pltpu.sync_copy(src_hbm.at[wid], src_v)
        for b in range(GRP):
            gather_start(b, b)
        pltpu.sync_copy(dst_hbm.at[wid], dst_v)
        pltpu.sync_copy(z2_hbm.at[pl.ds(rbase, ROWS_PER_SUB)],
                        agg_sh.at[pl.ds(rbase, ROWS_PER_SUB)])
        if with_cnt:
            pltpu.sync_copy(z1_hbm.at[pl.ds(rbase, ROWS_PER_SUB)], cntb_v)
            pltpu.sync_copy(cntb_v, cnt_sh.at[pl.ds(rbase, ROWS_PER_SUB)])
            pltpu.sync_copy(ones_hbm, ones_v)
        plsc.subcore_barrier()

        def group(g, carry):
            base = g * GRP
            for b in range(GRP):
                gather_wait(base + b, b)
                pltpu.async_copy(rows[b], agg_sh.at[dst_v.at[base + b]],
                                 ssem[b], add=True)
                if with_cnt:
                    pltpu.async_copy(ones_v, cnt_sh.at[dst_v.at[base + b]],
                                     csem[b], add=True)
            for b in range(GRP):
                pltpu.make_async_copy(rows[b], agg_sh.at[dst_v.at[base + b]],
                                      ssem[b]).wait()
                if with_cnt:
                    pltpu.make_async_copy(ones_v,
                                          cnt_sh.at[dst_v.at[base + b]],
                                          csem[b]).wait()
                gather_start(base + GRP + b, b)
            return carry

        lax.fori_loop(0, ngr, group, 0)
        for b in range(GRP):
            gather_wait(ngr * GRP + b, b)
        plsc.subcore_barrier()
        pltpu.sync_copy(agg_sh.at[pl.ds(rbase, ROWS_PER_SUB)],
                        out_hbm.at[cid, pl.ds(rbase, ROWS_PER_SUB)])
        if with_cnt:
            pltpu.sync_copy(cnt_sh.at[pl.ds(rbase, ROWS_PER_SUB)], cntb_v)
            pltpu.sync_copy(cntb_v,
                            cnt_hbm.at[pl.ds(cid * N_PAD + rbase,
                                             ROWS_PER_SUB)])

    return pl.kernel(body, out_type=out_type, mesh=mesh,
                     scratch_types=scratch,
                     compiler_params=pltpu.CompilerParams(
                         use_tc_tiling_on_sc=False))


def _tc_pre(x_ref, wl_ref, wr_ref, bl_ref, p_ref, r_ref):
    x = x_ref[...]
    p_ref[:N_NODES, :] = jnp.dot(x, wl_ref[...],
                                 preferred_element_type=jnp.float32)
    p_ref[N_NODES:, :] = jnp.zeros((PAD_ROWS, HID), jnp.float32)
    r_ref[...] = jnp.dot(x, wr_ref[...],
                         preferred_element_type=jnp.float32) + bl_ref[...]


def _tc_mid_first(a_ref, cnt0_ref, cnt1_ref, r_ref, wl_ref, wr_ref, bl_ref,
                  h_ref, p_ref, rn_ref, inv_ref):
    cnt = cnt0_ref[:N_NODES, :] + cnt1_ref[:N_NODES, :]
    inv = 1.0 / jnp.maximum(cnt, 1.0)
    inv_ref[...] = inv
    agg = a_ref[0, :N_NODES, :] + a_ref[1, :N_NODES, :]
    h = agg * inv + r_ref[...]
    h_ref[...] = h
    p_ref[:N_NODES, :] = jnp.dot(h, wl_ref[...],
                                 preferred_element_type=jnp.float32)
    p_ref[N_NODES:, :] = jnp.zeros((PAD_ROWS, HID), jnp.float32)
    rn_ref[...] = jnp.dot(h, wr_ref[...],
                          preferred_element_type=jnp.float32) + bl_ref[...]


def _tc_mid(a_ref, inv_ref, r_ref, wl_ref, wr_ref, bl_ref,
            h_ref, p_ref, rn_ref):
    agg = a_ref[0, :N_NODES, :] + a_ref[1, :N_NODES, :]
    h = agg * inv_ref[...] + r_ref[...]
    h_ref[...] = h
    p_ref[:N_NODES, :] = jnp.dot(h, wl_ref[...],
                                 preferred_element_type=jnp.float32)
    p_ref[N_NODES:, :] = jnp.zeros((PAD_ROWS, HID), jnp.float32)
    rn_ref[...] = jnp.dot(h, wr_ref[...],
                          preferred_element_type=jnp.float32) + bl_ref[...]


def _tc_final(a_ref, inv_ref, r_ref, h1_ref, h2_ref, h3_ref, batch_ref,
              fc1w_ref, fc1b_ref, fc2w_ref, fc2b_ref, out_ref):
    agg = a_ref[0, :N_NODES, :] + a_ref[1, :N_NODES, :]
    h4 = agg * inv_ref[...] + r_ref[...]
    b = batch_ref[...]
    gids = lax.broadcasted_iota(jnp.int32, (G_POOL, N_NODES), 0)
    mask = (b == gids).astype(jnp.float32)
    s1 = jnp.dot(mask, h1_ref[...], preferred_element_type=jnp.float32)
    s2 = jnp.dot(mask, h2_ref[...], preferred_element_type=jnp.float32)
    s3 = jnp.dot(mask, h3_ref[...], preferred_element_type=jnp.float32)
    s4 = jnp.dot(mask, h4, preferred_element_type=jnp.float32)
    cnt = jnp.sum(mask, axis=1, keepdims=True)
    pooled = jnp.concatenate([s1, s2, s3, s4], axis=1) / jnp.maximum(cnt, 1.0)
    hh = jnp.maximum(
        jnp.dot(pooled, fc1w_ref[...], preferred_element_type=jnp.float32)
        + fc1b_ref[...], 0.0)
    out_ref[...] = jnp.dot(hh, fc2w_ref[...],
                           preferred_element_type=jnp.float32) + fc2b_ref[...]


def _tc_call(body, out_shapes):
    return pl.pallas_call(body, out_shape=out_shapes)


def kernel(x, edge_index, batch, Wl0, bl0, Wr0, Wl1, bl1, Wr1, Wl2, bl2, Wr2,
           Wl3, bl3, Wr3, fc1_w, fc1_b, fc2_w, fc2_b):
    E = edge_index.shape[1]
    nch = -(-E // (NW * CHUNK))
    nch = -(-nch // GRP) * GRP
    epad = nch * NW * CHUNK
    pad = epad - E
    src = edge_index[0]
    dst = edge_index[1]
    if pad:
        pad_idx = (N_NODES
                   + (jnp.arange(pad, dtype=jnp.int32) % PAD_ROWS))
        src = jnp.concatenate([src.astype(jnp.int32), pad_idx])
        dst = jnp.concatenate([dst.astype(jnp.int32), pad_idx])
    junk = N_NODES + (jnp.arange(CHUNK, dtype=jnp.int32) % PAD_ROWS)
    junk3 = jnp.broadcast_to(junk, (NW, GRP, CHUNK))
    src3 = jnp.concatenate([src.reshape(NW, nch, CHUNK), junk3], axis=1)
    dst3 = jnp.concatenate([dst.reshape(NW, nch, CHUNK), junk3], axis=1)

    zeros2d = jnp.zeros((N_PAD, HID), jnp.float32)
    zeros1d = jnp.zeros((N_PAD,), jnp.float32)
    ones_c = jnp.ones((CHUNK,), jnp.float32)

    nhid = jax.ShapeDtypeStruct((N_NODES, HID), jnp.float32)
    phid = jax.ShapeDtypeStruct((N_PAD, HID), jnp.float32)
    ninv = jax.ShapeDtypeStruct((N_NODES, 1), jnp.float32)

    sc_first = _sc_agg(nch, with_cnt=True)
    sc_rest = _sc_agg(nch, with_cnt=False)

    p0, r0 = _tc_call(_tc_pre, [phid, nhid])(
        x, Wl0, Wr0, bl0.reshape(1, HID))
    a0, cntp = sc_first(src3, dst3, p0, zeros2d, ones_c, zeros1d)
    cntp = cntp.reshape(NUM_CORES, N_PAD)
    cnt0 = cntp[0].reshape(N_PAD, 1)
    cnt1 = cntp[1].reshape(N_PAD, 1)
    h1, p1, r1, inv = _tc_call(_tc_mid_first, [nhid, phid, nhid, ninv])(
        a0, cnt0, cnt1, r0, Wl1, Wr1, bl1.reshape(1, HID))
    a1 = sc_rest(src3, dst3, p1, zeros2d)
    h2, p2, r2 = _tc_call(_tc_mid, [nhid, phid, nhid])(
        a1, inv, r1, Wl2, Wr2, bl2.reshape(1, HID))
    a2 = sc_rest(src3, dst3, p2, zeros2d)
    h3, p3, r3 = _tc_call(_tc_mid, [nhid, phid, nhid])(
        a2, inv, r2, Wl3, Wr3, bl3.reshape(1, HID))
    a3 = sc_rest(src3, dst3, p3, zeros2d)
    out = _tc_call(_tc_final,
                   jax.ShapeDtypeStruct((G_POOL, fc2_w.shape[1]),
                                        jnp.float32))(
        a3, inv, r3, h1, h2, h3, batch.astype(jnp.int32).reshape(1, N_NODES),
        fc1_w, fc1_b.reshape(1, HID), fc2_w, fc2_b.reshape(1, fc2_w.shape[1]))
    return out


# speedup vs baseline: 12.5831x; 1.0234x over previous
import jax
import jax.numpy as jnp
from jax import lax
from jax.experimental import pallas as pl
from jax.experimental.pallas import tpu as pltpu
from jax.experimental.pallas import tpu_sc as plsc

N_NODES = 10000
PAD_ROWS = 112
N_PAD = N_NODES + PAD_ROWS
HID = 64
G_POOL = 64
NUM_CORES = 2
NUM_SUBCORES = 16
NW = NUM_CORES * NUM_SUBCORES
CHUNK = 128
GRP = 8
ROWS_PER_SUB = N_PAD // NUM_SUBCORES


def _sc_agg(nch, with_cnt):
    assert nch % GRP == 0
    mesh = plsc.VectorSubcoreMesh(
        core_axis_name="c", subcore_axis_name="s",
        num_cores=NUM_CORES, num_subcores=NUM_SUBCORES)
    agg_type = jax.ShapeDtypeStruct((NUM_CORES, N_PAD, HID), jnp.float32)
    if with_cnt:
        out_type = [agg_type,
                    jax.ShapeDtypeStruct((NUM_CORES * N_PAD,), jnp.float32)]
    else:
        out_type = agg_type
    scratch = [
        pltpu.VMEM((nch + GRP, CHUNK), jnp.int32),
        pltpu.VMEM((nch + GRP, CHUNK), jnp.int32),
        [pltpu.VMEM((CHUNK, HID), jnp.float32) for _ in range(GRP)],
        pltpu.VMEM_SHARED((N_PAD, HID), jnp.float32),
        pltpu.VMEM_SHARED((N_PAD,), jnp.float32),
        pltpu.VMEM((CHUNK,), jnp.float32),
        pltpu.VMEM((ROWS_PER_SUB,), jnp.float32),
        [pltpu.SemaphoreType.DMA for _ in range(GRP)],
        [pltpu.SemaphoreType.DMA for _ in range(GRP)],
        [pltpu.SemaphoreType.DMA for _ in range(GRP)],
    ]

    def body(*refs):
        if with_cnt:
            (src_hbm, dst_hbm, p_hbm, z2_hbm, ones_hbm, z1_hbm,
             out_hbm, cnt_hbm,
             src_v, dst_v, rows, agg_sh, cnt_sh, ones_v, cntb_v,
             gsem, ssem, csem) = refs
        else:
            (src_hbm, dst_hbm, p_hbm, z2_hbm,
             out_hbm,
             src_v, dst_v, rows, agg_sh, cnt_sh, ones_v, cntb_v,
             gsem, ssem, csem) = refs
        cid = lax.axis_index("c")
        sid = lax.axis_index("s")
        wid = cid * NUM_SUBCORES + sid
        rbase = sid * ROWS_PER_SUB
        def gather_start(j, b):
            pltpu.async_copy(p_hbm.at[src_v.at[j]], rows[b], gsem[b])

        def gather_wait(j, b):
            pltpu.make_async_copy(p_hbm.at[src_v.at[j]], rows[b],
                                  gsem[b]).wait()

        ngr = nch // GRP
        pltpu.sync_copy(src_hbm.at[wid], src_v)
        for b in range(GRP):
            gather_start(b, b)
        pltpu.sync_copy(dst_hbm.at[wid], dst_v)
        pltpu.sync_copy(z2_hbm, agg_sh.at[pl.ds(rbase, ROWS_PER_SUB)])
        if with_cnt:
            pltpu.sync_copy(z1_hbm, cntb_v)
            pltpu.sync_copy(cntb_v, cnt_sh.at[pl.ds(rbase, ROWS_PER_SUB)])
            pltpu.sync_copy(ones_hbm, ones_v)
        plsc.subcore_barrier()

        def group(g, carry):
            base = g * GRP
            for b in range(GRP):
                gather_wait(base + b, b)
                pltpu.async_copy(rows[b], agg_sh.at[dst_v.at[base + b]],
                                 ssem[b], add=True)
                if with_cnt:
                    pltpu.async_copy(ones_v, cnt_sh.at[dst_v.at[base + b]],
                                     csem[b], add=True)
            for b in range(GRP):
                pltpu.make_async_copy(rows[b], agg_sh.at[dst_v.at[base + b]],
                                      ssem[b]).wait()
                if with_cnt:
                    pltpu.make_async_copy(ones_v,
                                          cnt_sh.at[dst_v.at[base + b]],
                                          csem[b]).wait()
                gather_start(base + GRP + b, b)
            return carry

        lax.fori_loop(0, ngr, group, 0)
        for b in range(GRP):
            gather_wait(ngr * GRP + b, b)
        plsc.subcore_barrier()
        pltpu.sync_copy(agg_sh.at[pl.ds(rbase, ROWS_PER_SUB)],
                        out_hbm.at[cid, pl.ds(rbase, ROWS_PER_SUB)])
        if with_cnt:
            pltpu.sync_copy(cnt_sh.at[pl.ds(rbase, ROWS_PER_SUB)], cntb_v)
            pltpu.sync_copy(cntb_v,
                            cnt_hbm.at[pl.ds(cid * N_PAD + rbase,
                                             ROWS_PER_SUB)])

    return pl.kernel(body, out_type=out_type, mesh=mesh,
                     scratch_types=scratch,
                     compiler_params=pltpu.CompilerParams(
                         use_tc_tiling_on_sc=False))


def _tc_pre(x_ref, wlr_ref, bl_ref, p_ref, r_ref):
    pr = jnp.dot(x_ref[...], wlr_ref[...],
                 preferred_element_type=jnp.float32)
    p_ref[:N_NODES, :] = pr[:, :HID]
    p_ref[N_NODES:, :] = jnp.zeros((PAD_ROWS, HID), jnp.float32)
    r_ref[...] = pr[:, HID:] + bl_ref[...]


def _tc_mid_first(a_ref, cnt0_ref, cnt1_ref, r_ref, wlr_ref, bl_ref,
                  h_ref, p_ref, rn_ref, inv_ref):
    cnt = cnt0_ref[:N_NODES, :] + cnt1_ref[:N_NODES, :]
    inv = 1.0 / jnp.maximum(cnt, 1.0)
    inv_ref[...] = inv
    agg = a_ref[0, :N_NODES, :] + a_ref[1, :N_NODES, :]
    h = agg * inv + r_ref[...]
    h_ref[...] = h
    pr = jnp.dot(h, wlr_ref[...], preferred_element_type=jnp.float32)
    p_ref[:N_NODES, :] = pr[:, :HID]
    p_ref[N_NODES:, :] = jnp.zeros((PAD_ROWS, HID), jnp.float32)
    rn_ref[...] = pr[:, HID:] + bl_ref[...]


def _tc_mid(a_ref, inv_ref, r_ref, wlr_ref, bl_ref,
            h_ref, p_ref, rn_ref):
    agg = a_ref[0, :N_NODES, :] + a_ref[1, :N_NODES, :]
    h = agg * inv_ref[...] + r_ref[...]
    h_ref[...] = h
    pr = jnp.dot(h, wlr_ref[...], preferred_element_type=jnp.float32)
    p_ref[:N_NODES, :] = pr[:, :HID]
    p_ref[N_NODES:, :] = jnp.zeros((PAD_ROWS, HID), jnp.float32)
    rn_ref[...] = pr[:, HID:] + bl_ref[...]


def _tc_final(a_ref, inv_ref, r_ref, h1_ref, h2_ref, h3_ref, batch_ref,
              fc1w_ref, fc1b_ref, fc2w_ref, fc2b_ref, out_ref):
    agg = a_ref[0, :N_NODES, :] + a_ref[1, :N_NODES, :]
    h4 = agg * inv_ref[...] + r_ref[...]
    b = batch_ref[...]
    gids = lax.broadcasted_iota(jnp.int32, (G_POOL, N_NODES), 0)
    mask = (b == gids).astype(jnp.float32)
    s1 = jnp.dot(mask, h1_ref[...], preferred_element_type=jnp.float32)
    s2 = jnp.dot(mask, h2_ref[...], preferred_element_type=jnp.float32)
    s3 = jnp.dot(mask, h3_ref[...], preferred_element_type=jnp.float32)
    s4 = jnp.dot(mask, h4, preferred_element_type=jnp.float32)
    cnt = jnp.sum(mask, axis=1, keepdims=True)
    pooled = jnp.concatenate([s1, s2, s3, s4], axis=1) / jnp.maximum(cnt, 1.0)
    hh = jnp.maximum(
        jnp.dot(pooled, fc1w_ref[...], preferred_element_type=jnp.float32)
        + fc1b_ref[...], 0.0)
    out_ref[...] = jnp.dot(hh, fc2w_ref[...],
                           preferred_element_type=jnp.float32) + fc2b_ref[...]


def _tc_call(body, out_shapes):
    return pl.pallas_call(body, out_shape=out_shapes)


def kernel(x, edge_index, batch, Wl0, bl0, Wr0, Wl1, bl1, Wr1, Wl2, bl2, Wr2,
           Wl3, bl3, Wr3, fc1_w, fc1_b, fc2_w, fc2_b):
    E = edge_index.shape[1]
    nch = -(-E // (NW * CHUNK))
    nch = -(-nch // GRP) * GRP
    epad = nch * NW * CHUNK
    pad = epad - E
    src = edge_index[0]
    dst = edge_index[1]
    if pad:
        pad_idx = (N_NODES
                   + (jnp.arange(pad, dtype=jnp.int32) % PAD_ROWS))
        src = jnp.concatenate([src.astype(jnp.int32), pad_idx])
        dst = jnp.concatenate([dst.astype(jnp.int32), pad_idx])
    junk = N_NODES + (jnp.arange(CHUNK, dtype=jnp.int32) % PAD_ROWS)
    junk3 = jnp.broadcast_to(junk, (NW, GRP, CHUNK))
    src3 = jnp.concatenate([src.reshape(NW, nch, CHUNK), junk3], axis=1)
    dst3 = jnp.concatenate([dst.reshape(NW, nch, CHUNK), junk3], axis=1)

    zeros2d = jnp.zeros((ROWS_PER_SUB, HID), jnp.float32)
    zeros1d = jnp.zeros((ROWS_PER_SUB,), jnp.float32)
    ones_c = jnp.ones((CHUNK,), jnp.float32)

    nhid = jax.ShapeDtypeStruct((N_NODES, HID), jnp.float32)
    phid = jax.ShapeDtypeStruct((N_PAD, HID), jnp.float32)
    ninv = jax.ShapeDtypeStruct((N_NODES, 1), jnp.float32)

    sc_first = _sc_agg(nch, with_cnt=True)
    sc_rest = _sc_agg(nch, with_cnt=False)

    Wlr0 = jnp.concatenate([Wl0, Wr0], axis=1)
    Wlr1 = jnp.concatenate([Wl1, Wr1], axis=1)
    Wlr2 = jnp.concatenate([Wl2, Wr2], axis=1)
    Wlr3 = jnp.concatenate([Wl3, Wr3], axis=1)

    p0, r0 = _tc_call(_tc_pre, [phid, nhid])(
        x, Wlr0, bl0.reshape(1, HID))
    a0, cntp = sc_first(src3, dst3, p0, zeros2d, ones_c, zeros1d)
    cntp = cntp.reshape(NUM_CORES, N_PAD)
    cnt0 = cntp[0].reshape(N_PAD, 1)
    cnt1 = cntp[1].reshape(N_PAD, 1)
    h1, p1, r1, inv = _tc_call(_tc_mid_first, [nhid, phid, nhid, ninv])(
        a0, cnt0, cnt1, r0, Wlr1, bl1.reshape(1, HID))
    a1 = sc_rest(src3, dst3, p1, zeros2d)
    h2, p2, r2 = _tc_call(_tc_mid, [nhid, phid, nhid])(
        a1, inv, r1, Wlr2, bl2.reshape(1, HID))
    a2 = sc_rest(src3, dst3, p2, zeros2d)
    h3, p3, r3 = _tc_call(_tc_mid, [nhid, phid, nhid])(
        a2, inv, r2, Wlr3, bl3.reshape(1, HID))
    a3 = sc_rest(src3, dst3, p3, zeros2d)
    out = _tc_call(_tc_final,
                   jax.ShapeDtypeStruct((G_POOL, fc2_w.shape[1]),
                                        jnp.float32))(
        a3, inv, r3, h1, h2, h3, batch.astype(jnp.int32).reshape(1, N_NODES),
        fc1_w, fc1_b.reshape(1, HID), fc2_w, fc2_b.reshape(1, fc2_w.shape[1]))
    return out


# speedup vs baseline: 14.6798x; 1.1666x over previous
import jax
import jax.numpy as jnp
from jax import lax
from jax.experimental import pallas as pl
from jax.experimental.pallas import tpu as pltpu
from jax.experimental.pallas import tpu_sc as plsc

N_NODES = 10000
PAD_ROWS = 112
N_PAD = N_NODES + PAD_ROWS
HID = 64
G_POOL = 64
NUM_CORES = 2
NUM_SUBCORES = 16
NW = NUM_CORES * NUM_SUBCORES
CHUNK = 128
GRP = 8
ROWS_PER_SUB = N_PAD // NUM_SUBCORES


def _sc_agg(nch, with_cnt):
    assert nch % GRP == 0
    mesh = plsc.VectorSubcoreMesh(
        core_axis_name="c", subcore_axis_name="s",
        num_cores=NUM_CORES, num_subcores=NUM_SUBCORES)
    agg_type = jax.ShapeDtypeStruct((NUM_CORES, N_PAD, HID), jnp.float32)
    if with_cnt:
        out_type = [agg_type,
                    jax.ShapeDtypeStruct((NUM_CORES * N_PAD,), jnp.float32)]
    else:
        out_type = agg_type
    scratch = [
        pltpu.VMEM((nch, CHUNK), jnp.int32),
        pltpu.VMEM((nch, CHUNK), jnp.int32),
        [pltpu.VMEM((CHUNK, HID), jnp.float32) for _ in range(GRP)],
        pltpu.VMEM_SHARED((N_PAD, HID), jnp.float32),
        pltpu.VMEM_SHARED((N_PAD,), jnp.float32),
        pltpu.VMEM((CHUNK,), jnp.float32),
        pltpu.VMEM((ROWS_PER_SUB,), jnp.float32),
        [pltpu.SemaphoreType.DMA for _ in range(GRP)],
        [pltpu.SemaphoreType.DMA for _ in range(GRP)],
        [pltpu.SemaphoreType.DMA for _ in range(GRP)],
    ]

    def body(*refs):
        if with_cnt:
            (src_hbm, dst_hbm, p_hbm, z2_hbm, ones_hbm, z1_hbm,
             out_hbm, cnt_hbm,
             src_v, dst_v, rows, agg_sh, cnt_sh, ones_v, cntb_v,
             gsem, ssem, csem) = refs
        else:
            (src_hbm, dst_hbm, p_hbm, z2_hbm,
             out_hbm,
             src_v, dst_v, rows, agg_sh, cnt_sh, ones_v, cntb_v,
             gsem, ssem, csem) = refs
        cid = lax.axis_index("c")
        sid = lax.axis_index("s")
        wid = cid * NUM_SUBCORES + sid
        rbase = sid * ROWS_PER_SUB
        def gather_start(j, b):
            jc = jnp.minimum(j, nch - 1)
            pltpu.async_copy(p_hbm.at[src_v.at[jc]], rows[b], gsem[b])

        def gather_wait(j, b):
            jc = jnp.minimum(j, nch - 1)
            pltpu.make_async_copy(p_hbm.at[src_v.at[jc]], rows[b],
                                  gsem[b]).wait()

        ngr = nch // GRP
        pltpu.sync_copy(src_hbm.at[wid], src_v)
        for b in range(GRP):
            gather_start(b, b)
        pltpu.sync_copy(dst_hbm.at[wid], dst_v)
        pltpu.sync_copy(z2_hbm, agg_sh.at[pl.ds(rbase, ROWS_PER_SUB)])
        if with_cnt:
            pltpu.sync_copy(z1_hbm, cntb_v)
            pltpu.sync_copy(cntb_v, cnt_sh.at[pl.ds(rbase, ROWS_PER_SUB)])
            pltpu.sync_copy(ones_hbm, ones_v)
        plsc.subcore_barrier()

        def group(g, carry):
            base = g * GRP
            for b in range(GRP):
                gather_wait(base + b, b)
                pltpu.async_copy(rows[b], agg_sh.at[dst_v.at[base + b]],
                                 ssem[b], add=True)
                if with_cnt:
                    pltpu.async_copy(ones_v, cnt_sh.at[dst_v.at[base + b]],
                                     csem[b], add=True)
            for b in range(GRP):
                pltpu.make_async_copy(rows[b], agg_sh.at[dst_v.at[base + b]],
                                      ssem[b]).wait()
                if with_cnt:
                    pltpu.make_async_copy(ones_v,
                                          cnt_sh.at[dst_v.at[base + b]],
                                          csem[b]).wait()
                gather_start(base + GRP + b, b)
            return carry

        lax.fori_loop(0, ngr, group, 0)
        for b in range(GRP):
            gather_wait(ngr * GRP + b, b)
        plsc.subcore_barrier()
        pltpu.sync_copy(agg_sh.at[pl.ds(rbase, ROWS_PER_SUB)],
                        out_hbm.at[cid, pl.ds(rbase, ROWS_PER_SUB)])
        if with_cnt:
            pltpu.sync_copy(cnt_sh.at[pl.ds(rbase, ROWS_PER_SUB)], cntb_v)
            pltpu.sync_copy(cntb_v,
                            cnt_hbm.at[pl.ds(cid * N_PAD + rbase,
                                             ROWS_PER_SUB)])

    return pl.kernel(body, out_type=out_type, mesh=mesh,
                     scratch_types=scratch,
                     compiler_params=pltpu.CompilerParams(
                         use_tc_tiling_on_sc=False))


def _tc_pre(x_ref, wlr_ref, bl_ref, p_ref, r_ref):
    pr = jnp.dot(x_ref[...], wlr_ref[...],
                 preferred_element_type=jnp.float32)
    p_ref[:N_NODES, :] = pr[:, :HID]
    p_ref[N_NODES:, :] = jnp.zeros((PAD_ROWS, HID), jnp.float32)
    r_ref[...] = pr[:, HID:] + bl_ref[...]


def _tc_mid_first(a_ref, cnt0_ref, cnt1_ref, r_ref, wlr_ref, bl_ref,
                  h_ref, p_ref, rn_ref, inv_ref):
    cnt = cnt0_ref[:N_NODES, :] + cnt1_ref[:N_NODES, :]
    inv = 1.0 / jnp.maximum(cnt, 1.0)
    inv_ref[...] = inv
    agg = a_ref[0, :N_NODES, :] + a_ref[1, :N_NODES, :]
    h = agg * inv + r_ref[...]
    h_ref[...] = h
    pr = jnp.dot(h, wlr_ref[...], preferred_element_type=jnp.float32)
    p_ref[:N_NODES, :] = pr[:, :HID]
    p_ref[N_NODES:, :] = jnp.zeros((PAD_ROWS, HID), jnp.float32)
    rn_ref[...] = pr[:, HID:] + bl_ref[...]


def _tc_mid(a_ref, inv_ref, r_ref, wlr_ref, bl_ref,
            h_ref, p_ref, rn_ref):
    agg = a_ref[0, :N_NODES, :] + a_ref[1, :N_NODES, :]
    h = agg * inv_ref[...] + r_ref[...]
    h_ref[...] = h
    pr = jnp.dot(h, wlr_ref[...], preferred_element_type=jnp.float32)
    p_ref[:N_NODES, :] = pr[:, :HID]
    p_ref[N_NODES:, :] = jnp.zeros((PAD_ROWS, HID), jnp.float32)
    rn_ref[...] = pr[:, HID:] + bl_ref[...]


def _tc_final(a_ref, inv_ref, r_ref, h1_ref, h2_ref, h3_ref, batch_ref,
              fc1w_ref, fc1b_ref, fc2w_ref, fc2b_ref, out_ref):
    agg = a_ref[0, :N_NODES, :] + a_ref[1, :N_NODES, :]
    h4 = agg * inv_ref[...] + r_ref[...]
    b = batch_ref[...]
    gids = lax.broadcasted_iota(jnp.int32, (G_POOL, N_NODES), 0)
    mask = (b == gids).astype(jnp.float32)
    s1 = jnp.dot(mask, h1_ref[...], preferred_element_type=jnp.float32)
    s2 = jnp.dot(mask, h2_ref[...], preferred_element_type=jnp.float32)
    s3 = jnp.dot(mask, h3_ref[...], preferred_element_type=jnp.float32)
    s4 = jnp.dot(mask, h4, preferred_element_type=jnp.float32)
    cnt = jnp.sum(mask, axis=1, keepdims=True)
    pooled = jnp.concatenate([s1, s2, s3, s4], axis=1) / jnp.maximum(cnt, 1.0)
    hh = jnp.maximum(
        jnp.dot(pooled, fc1w_ref[...], preferred_element_type=jnp.float32)
        + fc1b_ref[...], 0.0)
    out_ref[...] = jnp.dot(hh, fc2w_ref[...],
                           preferred_element_type=jnp.float32) + fc2b_ref[...]


def _tc_call(body, out_shapes):
    return pl.pallas_call(body, out_shape=out_shapes)


def kernel(x, edge_index, batch, Wl0, bl0, Wr0, Wl1, bl1, Wr1, Wl2, bl2, Wr2,
           Wl3, bl3, Wr3, fc1_w, fc1_b, fc2_w, fc2_b):
    E = edge_index.shape[1]
    nch = -(-E // (NW * CHUNK))
    nch = -(-nch // GRP) * GRP
    epad = nch * NW * CHUNK
    pad = epad - E
    src = edge_index[0]
    dst = edge_index[1]
    if pad:
        pad_idx = (N_NODES
                   + (jnp.arange(pad, dtype=jnp.int32) % PAD_ROWS))
        src = jnp.concatenate([src.astype(jnp.int32), pad_idx])
        dst = jnp.concatenate([dst.astype(jnp.int32), pad_idx])
    src3 = src.reshape(NW, nch, CHUNK)
    dst3 = dst.reshape(NW, nch, CHUNK)

    zeros2d = jnp.zeros((ROWS_PER_SUB, HID), jnp.float32)
    zeros1d = jnp.zeros((ROWS_PER_SUB,), jnp.float32)
    ones_c = jnp.ones((CHUNK,), jnp.float32)

    nhid = jax.ShapeDtypeStruct((N_NODES, HID), jnp.float32)
    phid = jax.ShapeDtypeStruct((N_PAD, HID), jnp.float32)
    ninv = jax.ShapeDtypeStruct((N_NODES, 1), jnp.float32)

    sc_first = _sc_agg(nch, with_cnt=True)
    sc_rest = _sc_agg(nch, with_cnt=False)

    Wlr0 = jnp.concatenate([Wl0, Wr0], axis=1)
    Wlr1 = jnp.concatenate([Wl1, Wr1], axis=1)
    Wlr2 = jnp.concatenate([Wl2, Wr2], axis=1)
    Wlr3 = jnp.concatenate([Wl3, Wr3], axis=1)

    p0, r0 = _tc_call(_tc_pre, [phid, nhid])(
        x, Wlr0, bl0.reshape(1, HID))
    a0, cntp = sc_first(src3, dst3, p0, zeros2d, ones_c, zeros1d)
    cntp = cntp.reshape(NUM_CORES, N_PAD)
    cnt0 = cntp[0].reshape(N_PAD, 1)
    cnt1 = cntp[1].reshape(N_PAD, 1)
    h1, p1, r1, inv = _tc_call(_tc_mid_first, [nhid, phid, nhid, ninv])(
        a0, cnt0, cnt1, r0, Wlr1, bl1.reshape(1, HID))
    a1 = sc_rest(src3, dst3, p1, zeros2d)
    h2, p2, r2 = _tc_call(_tc_mid, [nhid, phid, nhid])(
        a1, inv, r1, Wlr2, bl2.reshape(1, HID))
    a2 = sc_rest(src3, dst3, p2, zeros2d)
    h3, p3, r3 = _tc_call(_tc_mid, [nhid, phid, nhid])(
        a2, inv, r2, Wlr3, bl3.reshape(1, HID))
    a3 = sc_rest(src3, dst3, p3, zeros2d)
    out = _tc_call(_tc_final,
                   jax.ShapeDtypeStruct((G_POOL, fc2_w.shape[1]),
                                        jnp.float32))(
        a3, inv, r3, h1, h2, h3, batch.astype(jnp.int32).reshape(1, N_NODES),
        fc1_w, fc1_b.reshape(1, HID), fc2_w, fc2_b.reshape(1, fc2_w.shape[1]))
    return out
